# baseline (speedup 1.0000x reference)
import functools

import jax
import jax.numpy as jnp
from jax import lax
from jax.experimental import pallas as pl
from jax.experimental.pallas import tpu as pltpu
from jax.experimental.pallas import tpu_sc as plsc

NUM_LABELS = 1000000
EMBED_DIM = 32
INPUT_SIZE = 5
BATCH = 16384

_NC = 2
_NS = 16
_NW = _NC * _NS
_ROWS = BATCH * INPUT_SIZE
_RPW = _ROWS // _NW
_CHUNK = 128
_NCH = _RPW // _CHUNK
_GROUPS = _RPW // INPUT_SIZE


def _sc_gather(idx3, table, w160):
    mesh = plsc.VectorSubcoreMesh(core_axis_name="c", subcore_axis_name="s")

    @functools.partial(
        pl.kernel,
        mesh=mesh,
        compiler_params=pltpu.CompilerParams(use_tc_tiling_on_sc=False),
        out_type=jax.ShapeDtypeStruct((_ROWS, EMBED_DIM), jnp.float32),
        scratch_types=[
            pltpu.VMEM((_NCH, _CHUNK), jnp.int32),
            pltpu.VMEM((EMBED_DIM * INPUT_SIZE,), jnp.float32),
            pltpu.VMEM((_RPW, EMBED_DIM), jnp.float32),
            pltpu.SemaphoreType.DMA,
        ],
    )
    def k(idx_hbm, table_hbm, w_hbm, out_hbm, idx_v, w_v, rows_v, sem):
        wid = lax.axis_index("s") * _NC + lax.axis_index("c")
        base = wid * _RPW

        pltpu.sync_copy(idx_hbm.at[wid], idx_v)
        pltpu.sync_copy(w_hbm, w_v)

        copies = []
        for c in range(_NCH):
            copies.append(
                pltpu.async_copy(
                    table_hbm.at[idx_v.at[c]],
                    rows_v.at[pl.ds(c * _CHUNK, _CHUNK)],
                    sem,
                )
            )
        for cp in copies:
            cp.wait()

        wv = [
            [w_v[pl.ds(p * EMBED_DIM + h * 16, 16)] for h in range(2)]
            for p in range(INPUT_SIZE)
        ]

        def body(g, _):
            r0 = g * INPUT_SIZE
            for p in range(INPUT_SIZE):
                for h in range(2):
                    sl = pl.ds(h * 16, 16)
                    rows_v[r0 + p, sl] = rows_v[r0 + p, sl] * wv[p][h]
            return _

        lax.fori_loop(0, _GROUPS, body, 0)

        pltpu.sync_copy(rows_v, out_hbm.at[pl.ds(base, _RPW)])

    return k(idx3, table, w160)


def kernel(x, table, weight):
    idx3 = x.astype(jnp.int32).reshape(_NW, _NCH, _CHUNK)
    w160 = jnp.repeat(weight.astype(jnp.float32), EMBED_DIM)
    out = _sc_gather(idx3, table, w160)
    return out.reshape(BATCH, INPUT_SIZE * EMBED_DIM)
